# baseline (speedup 1.0000x reference)
import functools

import jax
import jax.numpy as jnp
from jax import lax
from jax.experimental import pallas as pl
from jax.experimental.pallas import tpu as pltpu
from jax.experimental.pallas import tpu_sc as plsc

_EPS = 1e-5

_NC = 2
_NS = 16
_NW = _NC * _NS
_CHUNK = 64


def _sc_gather(table, idx1d):
    n = idx1d.shape[0]
    d = table.shape[1]
    b_per_w = n // _NW
    n_chunks = b_per_w // _CHUNK
    mesh = plsc.VectorSubcoreMesh(core_axis_name="c", subcore_axis_name="s")

    @functools.partial(
        pl.kernel,
        out_type=jax.ShapeDtypeStruct((n, d), table.dtype),
        mesh=mesh,
        scratch_types=[
            pltpu.VMEM((b_per_w,), jnp.int32),
            pltpu.VMEM((_CHUNK, d), jnp.float32),
            pltpu.VMEM((_CHUNK, d), jnp.float32),
            pltpu.SemaphoreType.DMA,
            pltpu.SemaphoreType.DMA,
            pltpu.SemaphoreType.DMA,
            pltpu.SemaphoreType.DMA,
        ],
    )
    def gather_kernel(table_hbm, idx_hbm, out_hbm, idx_v, rows0, rows1,
                      gsem0, gsem1, osem0, osem1):
        wid = lax.axis_index("s") * _NC + lax.axis_index("c")
        base = wid * b_per_w
        pltpu.sync_copy(idx_hbm.at[pl.ds(base, b_per_w)], idx_v)
        bufs = (rows0, rows1)
        gsems = (gsem0, gsem1)
        osems = (osem0, osem1)
        gather_h = [None, None]
        store_h = [None, None]
        gather_h[0] = pltpu.async_copy(
            table_hbm.at[idx_v.at[pl.ds(0, _CHUNK)]], bufs[0], gsems[0]
        )
        for c in range(n_chunks):
            cur = c & 1
            gather_h[cur].wait()
            if c + 1 < n_chunks:
                nb = (c + 1) & 1
                if store_h[nb] is not None:
                    store_h[nb].wait()
                gather_h[nb] = pltpu.async_copy(
                    table_hbm.at[idx_v.at[pl.ds((c + 1) * _CHUNK, _CHUNK)]],
                    bufs[nb],
                    gsems[nb],
                )
            store_h[cur] = pltpu.async_copy(
                bufs[cur], out_hbm.at[pl.ds(base + c * _CHUNK, _CHUNK)], osems[cur]
            )
        for h in store_h:
            if h is not None:
                h.wait()

    return gather_kernel(table, idx1d)


def _ln_chunk_body(gath_ref, pos_ref, tt_ref, wtt_ref, lnw_ref, lnb_ref,
                   out_ref, *maybe_prev):
    x = gath_ref[...]
    tt = tt_ref[0].astype(jnp.float32)
    w0 = wtt_ref[0, :][None, :]
    w1 = wtt_ref[1, :][None, :]
    tte = w0 + tt * (w1 - w0)
    x = x + pos_ref[...] + tte
    mu = jnp.mean(x, axis=-1, keepdims=True)
    xc = x - mu
    var = jnp.mean(xc * xc, axis=-1, keepdims=True)
    y = xc * lax.rsqrt(var + _EPS)
    out_ref[0] = y * lnw_ref[...] + lnb_ref[...]


def _body_with_prev(prev_ref, gath_ref, pos_ref, tt_ref, wtt_ref, lnw_ref,
                    lnb_ref, out_ref):
    _ln_chunk_body(gath_ref, pos_ref, tt_ref, wtt_ref, lnw_ref, lnb_ref,
                   out_ref)


def _tc_chunk(prev, gathered_k, k, tt3, W_pos, W_token_type, lnw2, lnb2,
              batch, seq):
    d = gathered_k.shape[-1]
    rows_per_blk = 1024
    seq_blks = seq // rows_per_blk

    specs = [
        pl.BlockSpec((rows_per_blk, d), lambda j: (j, 0)),
        pl.BlockSpec((rows_per_blk, d), lambda j: (j, 0)),
        pl.BlockSpec((1, rows_per_blk, 1), lambda j: (k, j, 0)),
        pl.BlockSpec((2, d), lambda j: (0, 0)),
        pl.BlockSpec((1, d), lambda j: (0, 0)),
        pl.BlockSpec((1, d), lambda j: (0, 0)),
    ]
    out_spec = pl.BlockSpec((1, rows_per_blk, d), lambda j: (k, j, 0))
    out_shape = jax.ShapeDtypeStruct((batch, seq, d), gathered_k.dtype)
    cp = pltpu.CompilerParams(dimension_semantics=("parallel",))
    args = (gathered_k, W_pos, tt3, W_token_type, lnw2, lnb2)
    if prev is None:
        return pl.pallas_call(
            _ln_chunk_body,
            grid=(seq_blks,),
            in_specs=specs,
            out_specs=out_spec,
            out_shape=out_shape,
            compiler_params=cp,
        )(*args)
    return pl.pallas_call(
        _body_with_prev,
        grid=(seq_blks,),
        in_specs=[pl.BlockSpec(memory_space=pl.ANY)] + specs,
        out_specs=out_spec,
        out_shape=out_shape,
        input_output_aliases={0: 0},
        compiler_params=cp,
    )(prev, *args)


@jax.jit
def kernel(input_ids, token_type_ids, W_E, W_pos, W_token_type, ln_w, ln_b):
    batch, seq = input_ids.shape
    d = W_E.shape[1]
    ids = input_ids.astype(jnp.int32)
    tt3 = token_type_ids.reshape(batch, seq, 1)
    lnw2 = ln_w.reshape(1, d)
    lnb2 = ln_b.reshape(1, d)

    gathered = [_sc_gather(W_E, ids[k]) for k in range(batch)]
    out = None
    for k in range(batch):
        out = _tc_chunk(out, gathered[k], k, tt3, W_pos, W_token_type,
                        lnw2, lnb2, batch, seq)
    return out


# speedup vs baseline: 1.0953x; 1.0953x over previous
import functools

import jax
import jax.numpy as jnp
from jax import lax
from jax.experimental import pallas as pl
from jax.experimental.pallas import tpu as pltpu
from jax.experimental.pallas import tpu_sc as plsc

_EPS = 1e-5

_NC = 2
_NS = 16
_NW = _NC * _NS
_CHUNK = 64


def _sc_gather(table, idx1d):
    n = idx1d.shape[0]
    d = table.shape[1]
    b_per_w = n // _NW
    n_chunks = b_per_w // _CHUNK
    mesh = plsc.VectorSubcoreMesh(core_axis_name="c", subcore_axis_name="s")

    @functools.partial(
        pl.kernel,
        out_type=jax.ShapeDtypeStruct((n, d), table.dtype),
        mesh=mesh,
        scratch_types=[
            pltpu.VMEM((b_per_w,), jnp.int32),
            pltpu.VMEM((_CHUNK, d), jnp.float32),
            pltpu.VMEM((_CHUNK, d), jnp.float32),
            pltpu.SemaphoreType.DMA,
            pltpu.SemaphoreType.DMA,
            pltpu.SemaphoreType.DMA,
            pltpu.SemaphoreType.DMA,
        ],
    )
    def gather_kernel(table_hbm, idx_hbm, out_hbm, idx_v, rows0, rows1,
                      gsem0, gsem1, osem0, osem1):
        wid = lax.axis_index("s") * _NC + lax.axis_index("c")
        base = wid * b_per_w
        pltpu.sync_copy(idx_hbm.at[pl.ds(base, b_per_w)], idx_v)
        bufs = (rows0, rows1)
        gsems = (gsem0, gsem1)
        osems = (osem0, osem1)
        gather_h = [None, None]
        store_h = [None, None]
        gather_h[0] = pltpu.async_copy(
            table_hbm.at[idx_v.at[pl.ds(0, _CHUNK)]], bufs[0], gsems[0]
        )
        for c in range(n_chunks):
            cur = c & 1
            gather_h[cur].wait()
            if c + 1 < n_chunks:
                nb = (c + 1) & 1
                if store_h[nb] is not None:
                    store_h[nb].wait()
                gather_h[nb] = pltpu.async_copy(
                    table_hbm.at[idx_v.at[pl.ds((c + 1) * _CHUNK, _CHUNK)]],
                    bufs[nb],
                    gsems[nb],
                )
            store_h[cur] = pltpu.async_copy(
                bufs[cur], out_hbm.at[pl.ds(base + c * _CHUNK, _CHUNK)], osems[cur]
            )
        for h in store_h:
            if h is not None:
                h.wait()

    return gather_kernel(table, idx1d)


def _ln_chunk_body(gath_ref, pos_ref, tt_ref, wtt_ref, lnw_ref, lnb_ref,
                   out_ref, *maybe_prev):
    x = gath_ref[...]
    tt = tt_ref[0].astype(jnp.float32)
    w0 = wtt_ref[0, :][None, :]
    w1 = wtt_ref[1, :][None, :]
    tte = w0 + tt * (w1 - w0)
    x = x + pos_ref[...] + tte
    mu = jnp.mean(x, axis=-1, keepdims=True)
    xc = x - mu
    var = jnp.mean(xc * xc, axis=-1, keepdims=True)
    y = xc * lax.rsqrt(var + _EPS)
    out_ref[0] = y * lnw_ref[...] + lnb_ref[...]


def _body_with_prev(prev_ref, gath_ref, pos_ref, tt_ref, wtt_ref, lnw_ref,
                    lnb_ref, out_ref):
    _ln_chunk_body(gath_ref, pos_ref, tt_ref, wtt_ref, lnw_ref, lnb_ref,
                   out_ref)


def _tc_chunk(prev, gathered_k, k, cb, tt3, W_pos, W_token_type, lnw2, lnb2,
              batch, seq):
    d = gathered_k.shape[-1]
    rows_per_blk = 1024
    seq_blks = seq // rows_per_blk

    specs = [
        pl.BlockSpec((rows_per_blk, d), lambda j, b: (b * seq_blks + j, 0)),
        pl.BlockSpec((rows_per_blk, d), lambda j, b: (j, 0)),
        pl.BlockSpec((1, rows_per_blk, 1), lambda j, b: (k * cb + b, j, 0)),
        pl.BlockSpec((2, d), lambda j, b: (0, 0)),
        pl.BlockSpec((1, d), lambda j, b: (0, 0)),
        pl.BlockSpec((1, d), lambda j, b: (0, 0)),
    ]
    out_spec = pl.BlockSpec((1, rows_per_blk, d), lambda j, b: (k * cb + b, j, 0))
    out_shape = jax.ShapeDtypeStruct((batch, seq, d), gathered_k.dtype)
    cp = pltpu.CompilerParams(dimension_semantics=("parallel", "parallel"))
    args = (gathered_k, W_pos, tt3, W_token_type, lnw2, lnb2)
    if prev is None:
        return pl.pallas_call(
            _ln_chunk_body,
            grid=(seq_blks, cb),
            in_specs=specs,
            out_specs=out_spec,
            out_shape=out_shape,
            compiler_params=cp,
        )(*args)
    return pl.pallas_call(
        _body_with_prev,
        grid=(seq_blks, cb),
        in_specs=[pl.BlockSpec(memory_space=pl.ANY)] + specs,
        out_specs=out_spec,
        out_shape=out_shape,
        input_output_aliases={0: 0},
        compiler_params=cp,
    )(prev, *args)


_N_CHUNKS = 2


@jax.jit
def kernel(input_ids, token_type_ids, W_E, W_pos, W_token_type, ln_w, ln_b):
    batch, seq = input_ids.shape
    d = W_E.shape[1]
    cb = batch // _N_CHUNKS
    ids = input_ids.astype(jnp.int32).reshape(_N_CHUNKS, cb * seq)
    tt3 = token_type_ids.reshape(batch, seq, 1)
    lnw2 = ln_w.reshape(1, d)
    lnb2 = ln_b.reshape(1, d)

    gathered = [_sc_gather(W_E, ids[k]) for k in range(_N_CHUNKS)]
    out = None
    for k in range(_N_CHUNKS):
        out = _tc_chunk(out, gathered[k], k, cb, tt3, W_pos, W_token_type,
                        lnw2, lnb2, batch, seq)
    return out
